# baseline (speedup 1.0000x reference)
import functools

import jax
import jax.numpy as jnp
from jax import lax
from jax.experimental import pallas as pl
from jax.experimental.pallas import tpu as pltpu
from jax.experimental.pallas import tpu_sc as plsc

N = 10000
NF = 10240
EP = 327680
TNR = NF // 16
RH = EP // (32 * 128)
DEPTH = 8
NCH = RH // DEPTH
C1 = 48
C2 = 32

_mesh = plsc.VectorSubcoreMesh(core_axis_name="c", subcore_axis_name="s", num_cores=2)


def _hist_body(dst_hbm, out_hbm, dacc, ones_v, idx_v, buf_v, sem):
    c = lax.axis_index("c")
    s = lax.axis_index("s")
    for i in range(128 // 16):
        ones_v[pl.ds(i * 16, 16)] = jnp.ones((16,), jnp.float32)
    for i in range(TNR // 16):
        buf_v[pl.ds(i * 16, 16)] = jnp.zeros((16,), jnp.float32)
    pltpu.sync_copy(dst_hbm.at[pl.ds((c * 16 + s) * RH, RH)], idx_v)
    pltpu.sync_copy(buf_v, dacc.at[pl.ds(s * TNR, TNR)])
    plsc.subcore_barrier()

    def chunk(q, carry):
        row0 = q * DEPTH
        descs = []
        for i in range(DEPTH):
            descs.append(
                pltpu.async_copy(ones_v, dacc.at[idx_v.at[row0 + i]], sem, add=True)
            )
        for d in descs:
            d.wait()
        return carry

    lax.fori_loop(0, NCH, chunk, None)
    plsc.subcore_barrier()
    pltpu.sync_copy(dacc.at[pl.ds(s * TNR, TNR)], buf_v)
    pltpu.sync_copy(buf_v, out_hbm.at[c, pl.ds(s * TNR, TNR)])


def _agg_body(depth, g_hbm, src_hbm, dst_hbm, out_hbm, acc, src_v, dst_v, buf_v, gsem, ssem):
    blk = depth * 128
    nch = RH // depth
    c = lax.axis_index("c")
    s = lax.axis_index("s")
    wid = c * 16 + s
    pltpu.sync_copy(src_hbm.at[pl.ds(wid * RH, RH)], src_v)
    pltpu.sync_copy(dst_hbm.at[pl.ds(wid * RH, RH)], dst_v)
    pltpu.sync_copy(g_hbm.at[pl.ds(s * TNR, TNR)], buf_v.at[pl.ds(0, TNR)])
    pltpu.sync_copy(buf_v.at[pl.ds(0, TNR)], acc.at[pl.ds(s * TNR, TNR)])
    plsc.subcore_barrier()

    def g_desc(q, i, half):
        return pltpu.make_async_copy(
            g_hbm.at[src_v.at[q * depth + i]],
            buf_v.at[pl.ds(half * blk + i * 128, 128)],
            gsem,
        )

    def s_desc(q, i, half):
        return pltpu.make_async_copy(
            buf_v.at[pl.ds(half * blk + i * 128, 128)],
            acc.at[dst_v.at[q * depth + i]],
            ssem,
        )

    for i in range(depth):
        g_desc(0, i, 0).start()

    def body(q, carry):
        half = lax.rem(q, 2)
        nhalf = 1 - half
        for i in range(depth):
            g_desc(q, i, half).wait()

        @pl.when(q > 0)
        def _():
            for i in range(depth):
                s_desc(q - 1, i, nhalf).wait()

        @pl.when(q < nch - 1)
        def _():
            for i in range(depth):
                g_desc(q + 1, i, nhalf).start()

        for i in range(depth):
            s_desc(q, i, half).start(add=True)
        return carry

    lax.fori_loop(0, nch, body, None)
    for i in range(depth):
        s_desc(nch - 1, i, (nch - 1) % 2).wait()
    plsc.subcore_barrier()
    pltpu.sync_copy(acc.at[pl.ds(s * TNR, TNR)], buf_v.at[pl.ds(0, TNR)])
    pltpu.sync_copy(buf_v.at[pl.ds(0, TNR)], out_hbm.at[c, pl.ds(s * TNR, TNR)])


def _make_hist():
    return pl.kernel(
        _hist_body,
        out_type=jax.ShapeDtypeStruct((2, NF), jnp.float32),
        mesh=_mesh,
        scratch_types=[
            pltpu.VMEM_SHARED((NF,), jnp.float32),
            pltpu.VMEM((128,), jnp.float32),
            pltpu.VMEM((RH, 128), jnp.int32),
            pltpu.VMEM((TNR,), jnp.float32),
            pltpu.SemaphoreType.DMA,
        ],
    )


def _make_agg(ch, depth):
    return pl.kernel(
        functools.partial(_agg_body, depth),
        out_type=jax.ShapeDtypeStruct((2, NF, ch), jnp.float32),
        mesh=_mesh,
        scratch_types=[
            pltpu.VMEM_SHARED((NF, ch), jnp.float32),
            pltpu.VMEM((RH, 128), jnp.int32),
            pltpu.VMEM((RH, 128), jnp.int32),
            pltpu.VMEM((max(2 * depth * 128, TNR), ch), jnp.float32),
            pltpu.SemaphoreType.DMA,
            pltpu.SemaphoreType.DMA,
        ],
        compiler_params=pltpu.CompilerParams(use_tc_tiling_on_sc=False),
    )


_R = 400
_G = N // _R


def _dinv_of(pt_blk):
    deg = jnp.sum(pt_blk, axis=1, keepdims=True) + 1.0
    return lax.rsqrt(deg)


def _mm1_body(x_ref, w_ref, pt_ref, o_ref):
    dinv = _dinv_of(pt_ref[...])
    v = jnp.dot(x_ref[...], w_ref[...], preferred_element_type=jnp.float32)
    o_ref[...] = v * dinv


def _mm2_body(sa_ref, g1_ref, pt_ref, b1_ref, w2_ref, o_ref):
    dinv = _dinv_of(pt_ref[...])
    h = sa_ref[0] + sa_ref[1] - g1_ref[...]
    r = jax.nn.relu(dinv * h + b1_ref[...])
    v = jnp.dot(r, w2_ref[...], preferred_element_type=jnp.float32)
    o_ref[...] = v * dinv


def _fin_body(sa_ref, g2_ref, pt_ref, b2_ref, o_ref):
    dinv = _dinv_of(pt_ref[...])
    o_ref[...] = (dinv * (sa_ref[0] + sa_ref[1] - g2_ref[...]) + b2_ref[...])[:, :20]


def _mm1(x, w1p, pt):
    return pl.pallas_call(
        _mm1_body,
        grid=(_G,),
        in_specs=[
            pl.BlockSpec((_R, 128), lambda i: (i, 0)),
            pl.BlockSpec((128, C1), lambda i: (0, 0)),
            pl.BlockSpec((_R, 2), lambda i: (i, 0)),
        ],
        out_specs=pl.BlockSpec((_R, C1), lambda i: (i, 0)),
        out_shape=jax.ShapeDtypeStruct((NF, C1), jnp.float32),
    )(x, w1p, pt)


def _mm2(sa, g1, pt, b1p, w2p):
    return pl.pallas_call(
        _mm2_body,
        grid=(_G,),
        in_specs=[
            pl.BlockSpec((2, _R, C1), lambda i: (0, i, 0)),
            pl.BlockSpec((_R, C1), lambda i: (i, 0)),
            pl.BlockSpec((_R, 2), lambda i: (i, 0)),
            pl.BlockSpec((1, C1), lambda i: (0, 0)),
            pl.BlockSpec((C1, C2), lambda i: (0, 0)),
        ],
        out_specs=pl.BlockSpec((_R, C2), lambda i: (i, 0)),
        out_shape=jax.ShapeDtypeStruct((NF, C2), jnp.float32),
    )(sa, g1, pt, b1p, w2p)


def _fin(sa, g2, pt, b2p):
    return pl.pallas_call(
        _fin_body,
        grid=(_G,),
        in_specs=[
            pl.BlockSpec((2, _R, C2), lambda i: (0, i, 0)),
            pl.BlockSpec((_R, C2), lambda i: (i, 0)),
            pl.BlockSpec((_R, 2), lambda i: (i, 0)),
            pl.BlockSpec((1, C2), lambda i: (0, 0)),
        ],
        out_specs=pl.BlockSpec((_R, 20), lambda i: (i, 0)),
        out_shape=jax.ShapeDtypeStruct((N, 20), jnp.float32),
    )(sa, g2, pt, b2p)


def kernel(x, edge_index, W1, b1, W2, b2):
    e = edge_index.shape[1]
    src = edge_index[0].astype(jnp.int32)
    dst = edge_index[1].astype(jnp.int32)
    pad = EP - e
    pidx = jnp.arange(pad, dtype=jnp.int32)
    src2 = jnp.concatenate([src, pidx]).reshape(EP // 128, 128)
    dst2 = jnp.concatenate([dst, N + (pidx & 127)]).reshape(EP // 128, 128)

    w1p = jnp.zeros((128, C1), jnp.float32).at[:, :40].set(W1)
    b1p = jnp.zeros((1, C1), jnp.float32).at[0, :40].set(b1)
    w2p = jnp.zeros((C1, C2), jnp.float32).at[:40, :20].set(W2)
    b2p = jnp.zeros((1, C2), jnp.float32).at[0, :20].set(b2)

    p2 = _make_hist()(dst2)
    pt = p2.T
    g1 = _mm1(x, w1p, pt)
    sa1 = _make_agg(C1, 5)(g1, src2, dst2)
    g2 = _mm2(sa1, g1, pt, b1p, w2p)
    sa2 = _make_agg(C2, 10)(g2, src2, dst2)
    return _fin(sa2, g2, pt, b2p)


# speedup vs baseline: 1.1570x; 1.1570x over previous
import functools

import jax
import jax.numpy as jnp
from jax import lax
from jax.experimental import pallas as pl
from jax.experimental.pallas import tpu as pltpu
from jax.experimental.pallas import tpu_sc as plsc

N = 10000
NF = 10240
EP = 327680
TNR = NF // 16
RH = EP // (32 * 128)
DEPTH = 8
NCH = RH // DEPTH
C1 = 48
C2 = 32

_mesh = plsc.VectorSubcoreMesh(core_axis_name="c", subcore_axis_name="s", num_cores=2)


def _hist_body(dst_hbm, out_hbm, dacc, ones_v, idx_v, buf_v, sem):
    c = lax.axis_index("c")
    s = lax.axis_index("s")
    for i in range(128 // 16):
        ones_v[pl.ds(i * 16, 16)] = jnp.ones((16,), jnp.float32)
    for i in range(TNR // 16):
        buf_v[pl.ds(i * 16, 16)] = jnp.zeros((16,), jnp.float32)
    pltpu.sync_copy(dst_hbm.at[pl.ds((c * 16 + s) * RH, RH)], idx_v)
    pltpu.sync_copy(buf_v, dacc.at[pl.ds(s * TNR, TNR)])
    plsc.subcore_barrier()

    def chunk(q, carry):
        row0 = q * DEPTH
        descs = []
        for i in range(DEPTH):
            descs.append(
                pltpu.async_copy(ones_v, dacc.at[idx_v.at[row0 + i]], sem, add=True)
            )
        for d in descs:
            d.wait()
        return carry

    lax.fori_loop(0, NCH, chunk, None)
    plsc.subcore_barrier()
    pltpu.sync_copy(dacc.at[pl.ds(s * TNR, TNR)], buf_v)
    pltpu.sync_copy(buf_v, out_hbm.at[c, pl.ds(s * TNR, TNR)])


def _agg_body(depth, g_hbm, src_hbm, dst_hbm, out_hbm, acc, src_v, dst_v, buf_v, gsem, ssem):
    blk = depth * 128
    nch = RH // depth
    c = lax.axis_index("c")
    s = lax.axis_index("s")
    wid = c * 16 + s
    pltpu.sync_copy(src_hbm.at[pl.ds(wid * RH, RH)], src_v)
    pltpu.sync_copy(dst_hbm.at[pl.ds(wid * RH, RH)], dst_v)
    pltpu.sync_copy(g_hbm.at[pl.ds(s * TNR, TNR)], buf_v.at[pl.ds(0, TNR)])
    pltpu.sync_copy(buf_v.at[pl.ds(0, TNR)], acc.at[pl.ds(s * TNR, TNR)])
    plsc.subcore_barrier()

    def g_desc(q, i, half):
        return pltpu.make_async_copy(
            g_hbm.at[src_v.at[q * depth + i]],
            buf_v.at[pl.ds(half * blk + i * 128, 128)],
            gsem,
        )

    def s_desc(q, i, half):
        return pltpu.make_async_copy(
            buf_v.at[pl.ds(half * blk + i * 128, 128)],
            acc.at[dst_v.at[q * depth + i]],
            ssem,
        )

    for i in range(depth):
        g_desc(0, i, 0).start()

    def body(q, carry):
        half = lax.rem(q, 2)
        nhalf = 1 - half
        for i in range(depth):
            g_desc(q, i, half).wait()

        @pl.when(q > 0)
        def _():
            for i in range(depth):
                s_desc(q - 1, i, nhalf).wait()

        @pl.when(q < nch - 1)
        def _():
            for i in range(depth):
                g_desc(q + 1, i, nhalf).start()

        for i in range(depth):
            s_desc(q, i, half).start(add=True)
        return carry

    lax.fori_loop(0, nch, body, None)
    for i in range(depth):
        s_desc(nch - 1, i, (nch - 1) % 2).wait()
    plsc.subcore_barrier()
    pltpu.sync_copy(acc.at[pl.ds(s * TNR, TNR)], buf_v.at[pl.ds(0, TNR)])
    pltpu.sync_copy(buf_v.at[pl.ds(0, TNR)], out_hbm.at[c, pl.ds(s * TNR, TNR)])


def _make_hist():
    return pl.kernel(
        _hist_body,
        out_type=jax.ShapeDtypeStruct((2, NF), jnp.float32),
        mesh=_mesh,
        scratch_types=[
            pltpu.VMEM_SHARED((NF,), jnp.float32),
            pltpu.VMEM((128,), jnp.float32),
            pltpu.VMEM((RH, 128), jnp.int32),
            pltpu.VMEM((TNR,), jnp.float32),
            pltpu.SemaphoreType.DMA,
        ],
    )


def _make_agg(ch, depth):
    return pl.kernel(
        functools.partial(_agg_body, depth),
        out_type=jax.ShapeDtypeStruct((2, NF, ch), jnp.float32),
        mesh=_mesh,
        scratch_types=[
            pltpu.VMEM_SHARED((NF, ch), jnp.float32),
            pltpu.VMEM((RH, 128), jnp.int32),
            pltpu.VMEM((RH, 128), jnp.int32),
            pltpu.VMEM((max(2 * depth * 128, TNR), ch), jnp.float32),
            pltpu.SemaphoreType.DMA,
            pltpu.SemaphoreType.DMA,
        ],
        compiler_params=pltpu.CompilerParams(use_tc_tiling_on_sc=False),
    )


_R = 2000
_G = N // _R


def _dinv_of(pt_blk):
    deg = jnp.sum(pt_blk, axis=1, keepdims=True) + 1.0
    return lax.rsqrt(deg)


def _mm1_body(x_ref, w_ref, pt_ref, o_ref):
    dinv = _dinv_of(pt_ref[...])
    v = jnp.dot(x_ref[...], w_ref[...], preferred_element_type=jnp.float32)
    o_ref[...] = v * dinv


def _mm2_body(sa_ref, g1_ref, pt_ref, b1_ref, w2_ref, o_ref):
    dinv = _dinv_of(pt_ref[...])
    h = sa_ref[0] + sa_ref[1] - g1_ref[...]
    r = jax.nn.relu(dinv * h + b1_ref[...])
    v = jnp.dot(r, w2_ref[...], preferred_element_type=jnp.float32)
    o_ref[...] = v * dinv


def _fin_body(sa_ref, g2_ref, pt_ref, b2_ref, o_ref):
    dinv = _dinv_of(pt_ref[...])
    o_ref[...] = (dinv * (sa_ref[0] + sa_ref[1] - g2_ref[...]) + b2_ref[...])[:, :20]


def _mm1(x, w1p, pt):
    return pl.pallas_call(
        _mm1_body,
        grid=(_G,),
        in_specs=[
            pl.BlockSpec((_R, 128), lambda i: (i, 0)),
            pl.BlockSpec((128, C1), lambda i: (0, 0)),
            pl.BlockSpec((_R, 2), lambda i: (i, 0)),
        ],
        out_specs=pl.BlockSpec((_R, C1), lambda i: (i, 0)),
        out_shape=jax.ShapeDtypeStruct((NF, C1), jnp.float32),
    )(x, w1p, pt)


def _mm2(sa, g1, pt, b1p, w2p):
    return pl.pallas_call(
        _mm2_body,
        grid=(_G,),
        in_specs=[
            pl.BlockSpec((2, _R, C1), lambda i: (0, i, 0)),
            pl.BlockSpec((_R, C1), lambda i: (i, 0)),
            pl.BlockSpec((_R, 2), lambda i: (i, 0)),
            pl.BlockSpec((1, C1), lambda i: (0, 0)),
            pl.BlockSpec((C1, C2), lambda i: (0, 0)),
        ],
        out_specs=pl.BlockSpec((_R, C2), lambda i: (i, 0)),
        out_shape=jax.ShapeDtypeStruct((NF, C2), jnp.float32),
    )(sa, g1, pt, b1p, w2p)


def _fin(sa, g2, pt, b2p):
    return pl.pallas_call(
        _fin_body,
        grid=(_G,),
        in_specs=[
            pl.BlockSpec((2, _R, C2), lambda i: (0, i, 0)),
            pl.BlockSpec((_R, C2), lambda i: (i, 0)),
            pl.BlockSpec((_R, 2), lambda i: (i, 0)),
            pl.BlockSpec((1, C2), lambda i: (0, 0)),
        ],
        out_specs=pl.BlockSpec((_R, 20), lambda i: (i, 0)),
        out_shape=jax.ShapeDtypeStruct((N, 20), jnp.float32),
    )(sa, g2, pt, b2p)


def kernel(x, edge_index, W1, b1, W2, b2):
    e = edge_index.shape[1]
    src = edge_index[0].astype(jnp.int32)
    dst = edge_index[1].astype(jnp.int32)
    pad = EP - e
    pidx = jnp.arange(pad, dtype=jnp.int32)
    src2 = jnp.concatenate([src, pidx]).reshape(EP // 128, 128)
    dst2 = jnp.concatenate([dst, N + (pidx & 127)]).reshape(EP // 128, 128)

    w1p = jnp.zeros((128, C1), jnp.float32).at[:, :40].set(W1)
    b1p = jnp.zeros((1, C1), jnp.float32).at[0, :40].set(b1)
    w2p = jnp.zeros((C1, C2), jnp.float32).at[:40, :20].set(W2)
    b2p = jnp.zeros((1, C2), jnp.float32).at[0, :20].set(b2)

    p2 = _make_hist()(dst2)
    pt = p2.T
    g1 = _mm1(x, w1p, pt)
    sa1 = _make_agg(C1, 5)(g1, src2, dst2)
    g2 = _mm2(sa1, g1, pt, b1p, w2p)
    sa2 = _make_agg(C2, 10)(g2, src2, dst2)
    return _fin(sa2, g2, pt, b2p)


# speedup vs baseline: 1.1610x; 1.0034x over previous
import functools

import jax
import jax.numpy as jnp
from jax import lax
from jax.experimental import pallas as pl
from jax.experimental.pallas import tpu as pltpu
from jax.experimental.pallas import tpu_sc as plsc

N = 10000
NF = 10240
EP = 327680
TNR = NF // 16
RH = EP // (32 * 128)
DEPTH = 8
NCH = RH // DEPTH
C1 = 48
C2 = 32

_mesh = plsc.VectorSubcoreMesh(core_axis_name="c", subcore_axis_name="s", num_cores=2)


def _hist_body(dst_hbm, out_hbm, dacc, ones_v, idx_v, buf_v, sem):
    c = lax.axis_index("c")
    s = lax.axis_index("s")
    for i in range(128 // 16):
        ones_v[pl.ds(i * 16, 16)] = jnp.ones((16,), jnp.float32)
    for i in range(TNR // 16):
        buf_v[pl.ds(i * 16, 16)] = jnp.zeros((16,), jnp.float32)
    pltpu.sync_copy(dst_hbm.at[pl.ds((c * 16 + s) * RH, RH)], idx_v)
    pltpu.sync_copy(buf_v, dacc.at[pl.ds(s * TNR, TNR)])
    plsc.subcore_barrier()

    def chunk(q, carry):
        row0 = q * DEPTH
        descs = []
        for i in range(DEPTH):
            descs.append(
                pltpu.async_copy(ones_v, dacc.at[idx_v.at[row0 + i]], sem, add=True)
            )
        for d in descs:
            d.wait()
        return carry

    lax.fori_loop(0, NCH, chunk, None)
    plsc.subcore_barrier()
    pltpu.sync_copy(dacc.at[pl.ds(s * TNR, TNR)], buf_v)
    pltpu.sync_copy(buf_v, out_hbm.at[c, pl.ds(s * TNR, TNR)])


def _agg_body(depth, g_hbm, src_hbm, dst_hbm, out_hbm, acc, src_v, dst_v, buf_v, gsem, ssem):
    blk = depth * 128
    nch = RH // depth
    c = lax.axis_index("c")
    s = lax.axis_index("s")
    wid = c * 16 + s
    pltpu.sync_copy(src_hbm.at[pl.ds(wid * RH, RH)], src_v)
    pltpu.sync_copy(dst_hbm.at[pl.ds(wid * RH, RH)], dst_v)
    pltpu.sync_copy(g_hbm.at[pl.ds(s * TNR, TNR)], buf_v.at[pl.ds(0, TNR)])
    pltpu.sync_copy(buf_v.at[pl.ds(0, TNR)], acc.at[pl.ds(s * TNR, TNR)])
    plsc.subcore_barrier()

    def g_desc(q, i, half):
        return pltpu.make_async_copy(
            g_hbm.at[src_v.at[q * depth + i]],
            buf_v.at[pl.ds(half * blk + i * 128, 128)],
            gsem,
        )

    def s_desc(q, i, half):
        return pltpu.make_async_copy(
            buf_v.at[pl.ds(half * blk + i * 128, 128)],
            acc.at[dst_v.at[q * depth + i]],
            ssem,
        )

    for i in range(depth):
        g_desc(0, i, 0).start()

    def body(q, carry):
        half = lax.rem(q, 2)
        nhalf = 1 - half
        for i in range(depth):
            g_desc(q, i, half).wait()

        @pl.when(q > 0)
        def _():
            for i in range(depth):
                s_desc(q - 1, i, nhalf).wait()

        @pl.when(q < nch - 1)
        def _():
            for i in range(depth):
                g_desc(q + 1, i, nhalf).start()

        for i in range(depth):
            s_desc(q, i, half).start(add=True)
        return carry

    lax.fori_loop(0, nch, body, None)
    for i in range(depth):
        s_desc(nch - 1, i, (nch - 1) % 2).wait()
    plsc.subcore_barrier()
    pltpu.sync_copy(acc.at[pl.ds(s * TNR, TNR)], buf_v.at[pl.ds(0, TNR)])
    pltpu.sync_copy(buf_v.at[pl.ds(0, TNR)], out_hbm.at[c, pl.ds(s * TNR, TNR)])


def _make_hist():
    return pl.kernel(
        _hist_body,
        out_type=jax.ShapeDtypeStruct((2, NF), jnp.float32),
        mesh=_mesh,
        scratch_types=[
            pltpu.VMEM_SHARED((NF,), jnp.float32),
            pltpu.VMEM((128,), jnp.float32),
            pltpu.VMEM((RH, 128), jnp.int32),
            pltpu.VMEM((TNR,), jnp.float32),
            pltpu.SemaphoreType.DMA,
        ],
    )


def _make_agg(ch, depth):
    return pl.kernel(
        functools.partial(_agg_body, depth),
        out_type=jax.ShapeDtypeStruct((2, NF, ch), jnp.float32),
        mesh=_mesh,
        scratch_types=[
            pltpu.VMEM_SHARED((NF, ch), jnp.float32),
            pltpu.VMEM((RH, 128), jnp.int32),
            pltpu.VMEM((RH, 128), jnp.int32),
            pltpu.VMEM((max(2 * depth * 128, TNR), ch), jnp.float32),
            pltpu.SemaphoreType.DMA,
            pltpu.SemaphoreType.DMA,
        ],
        compiler_params=pltpu.CompilerParams(use_tc_tiling_on_sc=False),
    )


_R = 10000
_G = N // _R


def _dinv_of(pt_blk):
    deg = jnp.sum(pt_blk, axis=1, keepdims=True) + 1.0
    return lax.rsqrt(deg)


def _mm1_body(x_ref, w_ref, pt_ref, o_ref):
    dinv = _dinv_of(pt_ref[...])
    v = jnp.dot(x_ref[...], w_ref[...], preferred_element_type=jnp.float32)
    o_ref[...] = v * dinv


def _mm2_body(sa_ref, g1_ref, pt_ref, b1_ref, w2_ref, o_ref):
    dinv = _dinv_of(pt_ref[...])
    h = sa_ref[0] + sa_ref[1] - g1_ref[...]
    r = jax.nn.relu(dinv * h + b1_ref[...])
    v = jnp.dot(r, w2_ref[...], preferred_element_type=jnp.float32)
    o_ref[...] = v * dinv


def _fin_body(sa_ref, g2_ref, pt_ref, b2_ref, o_ref):
    dinv = _dinv_of(pt_ref[...])
    o_ref[...] = (dinv * (sa_ref[0] + sa_ref[1] - g2_ref[...]) + b2_ref[...])[:, :20]


def _mm1(x, w1p, pt):
    return pl.pallas_call(
        _mm1_body,
        grid=(_G,),
        in_specs=[
            pl.BlockSpec((_R, 128), lambda i: (i, 0)),
            pl.BlockSpec((128, C1), lambda i: (0, 0)),
            pl.BlockSpec((_R, 2), lambda i: (i, 0)),
        ],
        out_specs=pl.BlockSpec((_R, C1), lambda i: (i, 0)),
        out_shape=jax.ShapeDtypeStruct((NF, C1), jnp.float32),
    )(x, w1p, pt)


def _mm2(sa, g1, pt, b1p, w2p):
    return pl.pallas_call(
        _mm2_body,
        grid=(_G,),
        in_specs=[
            pl.BlockSpec((2, _R, C1), lambda i: (0, i, 0)),
            pl.BlockSpec((_R, C1), lambda i: (i, 0)),
            pl.BlockSpec((_R, 2), lambda i: (i, 0)),
            pl.BlockSpec((1, C1), lambda i: (0, 0)),
            pl.BlockSpec((C1, C2), lambda i: (0, 0)),
        ],
        out_specs=pl.BlockSpec((_R, C2), lambda i: (i, 0)),
        out_shape=jax.ShapeDtypeStruct((NF, C2), jnp.float32),
    )(sa, g1, pt, b1p, w2p)


def _fin(sa, g2, pt, b2p):
    return pl.pallas_call(
        _fin_body,
        grid=(_G,),
        in_specs=[
            pl.BlockSpec((2, _R, C2), lambda i: (0, i, 0)),
            pl.BlockSpec((_R, C2), lambda i: (i, 0)),
            pl.BlockSpec((_R, 2), lambda i: (i, 0)),
            pl.BlockSpec((1, C2), lambda i: (0, 0)),
        ],
        out_specs=pl.BlockSpec((_R, 20), lambda i: (i, 0)),
        out_shape=jax.ShapeDtypeStruct((N, 20), jnp.float32),
    )(sa, g2, pt, b2p)


def kernel(x, edge_index, W1, b1, W2, b2):
    e = edge_index.shape[1]
    src = edge_index[0].astype(jnp.int32)
    dst = edge_index[1].astype(jnp.int32)
    pad = EP - e
    pidx = jnp.arange(pad, dtype=jnp.int32)
    src2 = jnp.concatenate([src, pidx]).reshape(EP // 128, 128)
    dst2 = jnp.concatenate([dst, N + (pidx & 127)]).reshape(EP // 128, 128)

    w1p = jnp.zeros((128, C1), jnp.float32).at[:, :40].set(W1)
    b1p = jnp.zeros((1, C1), jnp.float32).at[0, :40].set(b1)
    w2p = jnp.zeros((C1, C2), jnp.float32).at[:40, :20].set(W2)
    b2p = jnp.zeros((1, C2), jnp.float32).at[0, :20].set(b2)

    p2 = _make_hist()(dst2)
    pt = p2.T
    g1 = _mm1(x, w1p, pt)
    sa1 = _make_agg(C1, 5)(g1, src2, dst2)
    g2 = _mm2(sa1, g1, pt, b1p, w2p)
    sa2 = _make_agg(C2, 10)(g2, src2, dst2)
    return _fin(sa2, g2, pt, b2p)
